# Initial kernel scaffold; baseline (speedup 1.0000x reference)
import jax
import jax.numpy as jnp
from jax.experimental import pallas as pl


def kernel(x, edge_index, batch, params):
    raise NotImplementedError("write your pallas kernel here")


# speedup vs baseline: 4.2423x; 4.2423x over previous
import functools

import jax
import jax.numpy as jnp
from jax import lax
from jax.experimental import pallas as pl
from jax.experimental.pallas import tpu as pltpu
from jax.experimental.pallas import tpu_sc as plsc

_BN_EPS = 1e-5


def _make_sc_agg(n, h, e):
    ncores, nsub = 2, 16
    nw = ncores * nsub
    epw = e // nw
    ch = 80
    nch = epw // ch
    npad = -(-n // 128) * 128
    rpt = npad // nsub

    mesh = plsc.VectorSubcoreMesh(core_axis_name="c", subcore_axis_name="s")

    @functools.partial(
        pl.kernel,
        out_type=jax.ShapeDtypeStruct((ncores, npad, h), jnp.float32),
        mesh=mesh,
        scratch_types=[
            pltpu.VMEM((ch,), jnp.int32),
            pltpu.VMEM((ch,), jnp.int32),
            pltpu.VMEM((ch, h), jnp.float32),
            pltpu.VMEM((8, h), jnp.float32),
            pltpu.VMEM_SHARED((npad, h), jnp.float32),
            pltpu.SemaphoreType.DMA,
        ],
    )
    def agg_kernel(h_hbm, src_hbm, dst_hbm, out_hbm, src_v, dst_v, rows_v,
                   zbuf, acc, sem):
        cid = lax.axis_index("c")
        sid = lax.axis_index("s")

        for i in range(8):
            for j in range(h // 16):
                zbuf[i, pl.ds(j * 16, 16)] = jnp.zeros((16,), jnp.float32)

        def zcopy(k, carry):
            pltpu.sync_copy(zbuf, acc.at[pl.ds(sid * rpt + k * 8, 8)])
            return carry

        lax.fori_loop(0, rpt // 8, zcopy, 0)
        plsc.subcore_barrier()

        base = (cid * nsub + sid) * epw

        def chunk(i, carry):
            st = base + i * ch
            pltpu.sync_copy(src_hbm.at[pl.ds(st, ch)], src_v)
            pltpu.sync_copy(dst_hbm.at[pl.ds(st, ch)], dst_v)
            pltpu.async_copy(h_hbm.at[src_v], rows_v, sem).wait()
            pltpu.sync_copy(rows_v, acc.at[dst_v], add=True)
            return carry

        lax.fori_loop(0, nch, chunk, 0)
        plsc.subcore_barrier()
        pltpu.sync_copy(acc.at[pl.ds(sid * rpt, rpt)],
                        out_hbm.at[cid, pl.ds(sid * rpt, rpt)])

    return agg_kernel


def _mm_in_body(nblk, x_ref, w_ref, b_ref, t_ref, s_ref, acc):
    j = pl.program_id(0)
    t = jnp.dot(x_ref[...], w_ref[...], preferred_element_type=jnp.float32)
    t = t + b_ref[...]
    t_ref[...] = t
    blk = jnp.concatenate([jnp.sum(t, axis=0, keepdims=True),
                           jnp.sum(t * t, axis=0, keepdims=True)], axis=0)

    @pl.when(j == 0)
    def _():
        acc[...] = jnp.zeros_like(acc)

    acc[...] += blk

    @pl.when(j == nblk - 1)
    def _():
        s_ref[...] = acc[...]


def _mm_gin_body(nblk, h_ref, agg_ref, eps_ref, w_ref, b_ref, t_ref, s_ref,
                 acc):
    j = pl.program_id(0)
    a = agg_ref[...]
    pre = (1.0 + eps_ref[0, 0]) * h_ref[...] + a[0] + a[1]
    t = jnp.dot(pre, w_ref[...], preferred_element_type=jnp.float32)
    t = t + b_ref[...]
    t_ref[...] = t
    blk = jnp.concatenate([jnp.sum(t, axis=0, keepdims=True),
                           jnp.sum(t * t, axis=0, keepdims=True)], axis=0)

    @pl.when(j == 0)
    def _():
        acc[...] = jnp.zeros_like(acc)

    acc[...] += blk

    @pl.when(j == nblk - 1)
    def _():
        s_ref[...] = acc[...]


def _bn_scale_shift(s, g, be, n_total):
    m = s[0:1, :] / n_total
    v = s[1:2, :] / n_total - m * m
    scale = g * lax.rsqrt(v + _BN_EPS)
    shift = be - m * scale
    return scale, shift


def _bn_mm_body(nblk, n_total, t1_ref, s1_ref, g_ref, be_ref, w_ref, b_ref,
                t2_ref, s2_ref, acc):
    j = pl.program_id(0)
    scale, shift = _bn_scale_shift(s1_ref[...], g_ref[...], be_ref[...],
                                   n_total)
    u = jnp.maximum(t1_ref[...] * scale + shift, 0.0)
    t2 = jnp.dot(u, w_ref[...], preferred_element_type=jnp.float32)
    t2 = t2 + b_ref[...]
    t2_ref[...] = t2
    blk = jnp.concatenate([jnp.sum(t2, axis=0, keepdims=True),
                           jnp.sum(t2 * t2, axis=0, keepdims=True)], axis=0)

    @pl.when(j == 0)
    def _():
        acc[...] = jnp.zeros_like(acc)

    acc[...] += blk

    @pl.when(j == nblk - 1)
    def _():
        s2_ref[...] = acc[...]


def _bn_relu_body(n_total, t_ref, s_ref, g_ref, be_ref, h_ref):
    scale, shift = _bn_scale_shift(s_ref[...], g_ref[...], be_ref[...],
                                   n_total)
    h_ref[...] = jnp.maximum(t_ref[...] * scale + shift, 0.0)


def _bn_relu_res_body(n_total, t_ref, s_ref, g_ref, be_ref, prev_ref, h_ref):
    scale, shift = _bn_scale_shift(s_ref[...], g_ref[...], be_ref[...],
                                   n_total)
    h_ref[...] = jnp.maximum(t_ref[...] * scale + shift, 0.0) + prev_ref[...]


def _pool_body(nblk, g_seg, h_ref, b3_ref, wc1_ref, bc1_ref, wc2_ref, bc2_ref,
               wcf_ref, bcf_ref, logits_ref, conf_ref, sums_acc, cnts_acc):
    j = pl.program_id(0)
    bvec = b3_ref[0, 0, :]
    onehot = (bvec[:, None] == lax.broadcasted_iota(jnp.int32, (1, 128), 1))
    onehot = onehot.astype(jnp.float32)
    hb = h_ref[...]
    part = lax.dot_general(onehot, hb, (((0,), (0,)), ((), ())),
                           preferred_element_type=jnp.float32)
    cnt = lax.dot_general(onehot, jnp.ones_like(hb), (((0,), (0,)), ((), ())),
                          preferred_element_type=jnp.float32)

    @pl.when(j == 0)
    def _():
        sums_acc[...] = jnp.zeros_like(sums_acc)
        cnts_acc[...] = jnp.zeros_like(cnts_acc)

    sums_acc[...] += part
    cnts_acc[...] += cnt

    @pl.when(j == nblk - 1)
    def _():
        sums = sums_acc[0:g_seg, :]
        cnts = jnp.maximum(cnts_acc[0:g_seg, :], 1.0)
        mean = sums / cnts
        emb = jnp.concatenate([mean, sums], axis=1)
        hm = jnp.dot(emb, wc1_ref[...], preferred_element_type=jnp.float32)
        hm = jnp.maximum(hm + bc1_ref[...], 0.0)
        logits_ref[...] = jnp.dot(
            hm, wc2_ref[...], preferred_element_type=jnp.float32) + bc2_ref[...]
        cf = jnp.dot(emb, wcf_ref[...], preferred_element_type=jnp.float32)
        conf_ref[...] = jax.nn.sigmoid(cf + bcf_ref[...])


def _row_spec(blk, width):
    return pl.BlockSpec((blk, width), lambda j: (j, 0))


def _full_spec(shape):
    nd = len(shape)
    return pl.BlockSpec(shape, lambda j: (0,) * nd)


def _mm_in(x, w, b, blk):
    n, _ = x.shape
    k = w.shape[1]
    nblk = n // blk
    return pl.pallas_call(
        functools.partial(_mm_in_body, nblk),
        grid=(nblk,),
        in_specs=[_row_spec(blk, x.shape[1]), _full_spec(w.shape),
                  _full_spec((1, k))],
        out_specs=[_row_spec(blk, k), _full_spec((2, k))],
        out_shape=[jax.ShapeDtypeStruct((n, k), jnp.float32),
                   jax.ShapeDtypeStruct((2, k), jnp.float32)],
        scratch_shapes=[pltpu.VMEM((2, k), jnp.float32)],
    )(x, w, b.reshape(1, k))


def _mm_gin(hcur, aggp, eps, w, b, blk):
    n, hdim = hcur.shape
    k = w.shape[1]
    nblk = n // blk
    return pl.pallas_call(
        functools.partial(_mm_gin_body, nblk),
        grid=(nblk,),
        in_specs=[_row_spec(blk, hdim),
                  pl.BlockSpec((2, blk, hdim), lambda j: (0, j, 0)),
                  pl.BlockSpec(memory_space=pltpu.SMEM),
                  _full_spec(w.shape), _full_spec((1, k))],
        out_specs=[_row_spec(blk, k), _full_spec((2, k))],
        out_shape=[jax.ShapeDtypeStruct((n, k), jnp.float32),
                   jax.ShapeDtypeStruct((2, k), jnp.float32)],
        scratch_shapes=[pltpu.VMEM((2, k), jnp.float32)],
    )(hcur, aggp, eps.reshape(1, 1), w, b.reshape(1, k))


def _bn_mm(t1, s1, g, be, w, b, blk):
    n, k1 = t1.shape
    k2 = w.shape[1]
    nblk = n // blk
    return pl.pallas_call(
        functools.partial(_bn_mm_body, nblk, float(n)),
        grid=(nblk,),
        in_specs=[_row_spec(blk, k1), _full_spec((2, k1)),
                  _full_spec((1, k1)), _full_spec((1, k1)),
                  _full_spec(w.shape), _full_spec((1, k2))],
        out_specs=[_row_spec(blk, k2), _full_spec((2, k2))],
        out_shape=[jax.ShapeDtypeStruct((n, k2), jnp.float32),
                   jax.ShapeDtypeStruct((2, k2), jnp.float32)],
        scratch_shapes=[pltpu.VMEM((2, k2), jnp.float32)],
    )(t1, s1, g.reshape(1, k1), be.reshape(1, k1), w, b.reshape(1, k2))


def _bn_relu(t, s, g, be, blk):
    n, k = t.shape
    nblk = n // blk
    return pl.pallas_call(
        functools.partial(_bn_relu_body, float(n)),
        grid=(nblk,),
        in_specs=[_row_spec(blk, k), _full_spec((2, k)),
                  _full_spec((1, k)), _full_spec((1, k))],
        out_specs=_row_spec(blk, k),
        out_shape=jax.ShapeDtypeStruct((n, k), jnp.float32),
    )(t, s, g.reshape(1, k), be.reshape(1, k))


def _bn_relu_res(t, s, g, be, prev, blk):
    n, k = t.shape
    nblk = n // blk
    return pl.pallas_call(
        functools.partial(_bn_relu_res_body, float(n)),
        grid=(nblk,),
        in_specs=[_row_spec(blk, k), _full_spec((2, k)),
                  _full_spec((1, k)), _full_spec((1, k)),
                  _row_spec(blk, k)],
        out_specs=_row_spec(blk, k),
        out_shape=jax.ShapeDtypeStruct((n, k), jnp.float32),
    )(t, s, g.reshape(1, k), be.reshape(1, k), prev)


def _pool_cls(hcur, batch, g_seg, wc1, bc1, wc2p, bc2p, wcfp, bcfp, blk):
    n, hdim = hcur.shape
    nblk = n // blk
    b3 = batch.reshape(nblk, 1, blk)
    return pl.pallas_call(
        functools.partial(_pool_body, nblk, g_seg),
        grid=(nblk,),
        in_specs=[_row_spec(blk, hdim),
                  pl.BlockSpec((1, 1, blk), lambda j: (j, 0, 0)),
                  _full_spec(wc1.shape), _full_spec((1, wc1.shape[1])),
                  _full_spec(wc2p.shape), _full_spec((1, wc2p.shape[1])),
                  _full_spec(wcfp.shape), _full_spec((1, wcfp.shape[1]))],
        out_specs=[_full_spec((g_seg, wc2p.shape[1])),
                   _full_spec((g_seg, wcfp.shape[1]))],
        out_shape=[jax.ShapeDtypeStruct((g_seg, wc2p.shape[1]), jnp.float32),
                   jax.ShapeDtypeStruct((g_seg, wcfp.shape[1]), jnp.float32)],
        scratch_shapes=[pltpu.VMEM((128, hdim), jnp.float32),
                        pltpu.VMEM((128, hdim), jnp.float32)],
    )(hcur, b3, wc1, bc1.reshape(1, -1), wc2p, bc2p.reshape(1, -1),
      wcfp, bcfp.reshape(1, -1))


def kernel(x, edge_index, batch, params):
    p = params
    n, _ = x.shape
    e = edge_index.shape[1]
    hdim = p['W_in'].shape[1]
    g_seg = 64
    blk = 1000

    src = edge_index[0]
    dst = edge_index[1]

    sc_agg = _make_sc_agg(n, hdim, e)

    t0, s0 = _mm_in(x, p['W_in'], p['b_in'], blk)
    hcur = _bn_relu(t0, s0, p['g_bn_in'], p['be_bn_in'], blk)

    for l in range(3):
        aggp = sc_agg(hcur, src, dst)
        t1, s1 = _mm_gin(hcur, aggp, p['eps_%d' % l], p['W1_%d' % l],
                         p['b1_%d' % l], blk)
        t2, s2 = _bn_mm(t1, s1, p['g_mid_%d' % l], p['be_mid_%d' % l],
                        p['W2_%d' % l], p['b2_%d' % l], blk)
        hcur = _bn_relu_res(t2, s2, p['g_bn_%d' % l], p['be_bn_%d' % l],
                            hcur, blk)

    wc2p = jnp.pad(p['W_c2'], ((0, 0), (0, 128 - p['W_c2'].shape[1])))
    bc2p = jnp.pad(p['b_c2'], (0, 128 - p['b_c2'].shape[0]))
    wcfp = jnp.pad(p['W_conf'], ((0, 0), (0, 128 - p['W_conf'].shape[1])))
    bcfp = jnp.pad(p['b_conf'], (0, 128 - p['b_conf'].shape[0]))

    logits_pad, conf_pad = _pool_cls(hcur, batch, g_seg, p['W_c1'], p['b_c1'],
                                     wc2p, bc2p, wcfp, bcfp, blk)
    return logits_pad[:, :2], conf_pad[:, :1]
